# baseline (speedup 1.0000x reference)
import functools

import jax
import jax.numpy as jnp
from jax import lax
from jax.experimental import pallas as pl
from jax.experimental.pallas import tpu as pltpu
from jax.experimental.pallas import tpu_sc as plsc

B = 16384
KF = 32
FL = 64
N = 1000000


def _dot0(a, b):
    return jax.lax.dot_general(a, b, (((0,), (0,)), ((), ())),
                               preferred_element_type=jnp.float32)


def _round_bf16_bits(v):
    b = jax.lax.bitcast_convert_type(v, jnp.uint32)
    return (b + 0x7FFF + ((b >> 16) & 1)) >> 16


def _tx_body(u_r, i_r, g_r, M_r, brow_r, mrow_r, x_r):
    bn = u_r.shape[1]
    cat = jnp.concatenate([u_r[...], i_r[...], g_r[...]], axis=0)
    h = _dot0(cat, M_r[...]) + brow_r[...]
    x = jnp.where(mrow_r[...] > 0, jnp.maximum(h, 0.0), h)
    be = _round_bf16_bits(x[:bn // 2])
    bo = _round_bf16_bits(x[bn // 2:])
    x_r[...] = be | (bo << 16)


def _tx_call(utT, itT, gT, M, brow, mrow, *, bn=4096):
    grid = (pl.cdiv(N, bn),)
    return pl.pallas_call(
        _tx_body,
        grid=grid,
        in_specs=[
            pl.BlockSpec((KF, bn), lambda i: (0, i)),
            pl.BlockSpec((KF, bn), lambda i: (0, i)),
            pl.BlockSpec((FL, bn), lambda i: (0, i)),
            pl.BlockSpec((4 * KF, 4 * KF), lambda i: (0, 0)),
            pl.BlockSpec((1, 4 * KF), lambda i: (0, 0)),
            pl.BlockSpec((1, 4 * KF), lambda i: (0, 0)),
        ],
        out_specs=pl.BlockSpec((bn // 2, 4 * KF), lambda i: (i, 0)),
        out_shape=jax.ShapeDtypeStruct(
            (pl.cdiv(N, bn) * (bn // 2), 4 * KF), jnp.uint32),
    )(utT, itT, gT, M, brow, mrow)


@functools.cache
def _sc_gather():
    info = plsc.get_sparse_core_info()
    nw = info.num_cores * info.num_subcores
    bpw = B // nw
    half = bpw // 2
    D = 2 * KF + FL
    mesh = plsc.VectorSubcoreMesh(core_axis_name="c", subcore_axis_name="s")

    @functools.partial(
        pl.kernel,
        mesh=mesh,
        out_type=[
            jax.ShapeDtypeStruct((B, D), jnp.uint32),
            jax.ShapeDtypeStruct((B, D), jnp.uint32),
        ],
        scratch_types=[
            pltpu.VMEM((bpw,), jnp.int32),
            pltpu.VMEM((bpw,), jnp.int32),
            pltpu.VMEM((half, D), jnp.uint32),
            pltpu.VMEM((half, D), jnp.uint32),
            pltpu.SemaphoreType.DMA,
            pltpu.SemaphoreType.DMA,
        ],
    )
    def gather_k(uids, iids, x, xu_o, xi_o,
                 uidx, iidx, buf_a, buf_b, sem_a, sem_b):
        wid = lax.axis_index("s") * info.num_cores + lax.axis_index("c")
        base = wid * bpw
        pltpu.sync_copy(uids.at[pl.ds(base, bpw)], uidx)
        pltpu.sync_copy(iids.at[pl.ds(base, bpw)], iidx)

        @pl.loop(0, 2)
        def _(r):
            ca = pltpu.async_copy(
                x.at[uidx.at[pl.ds(r * half, half)]], buf_a, sem_a)
            cb = pltpu.async_copy(
                x.at[iidx.at[pl.ds(r * half, half)]], buf_b, sem_b)
            ca.wait()
            cb.wait()
            pltpu.sync_copy(buf_a, xu_o.at[pl.ds(base + r * half, half)])
            pltpu.sync_copy(buf_b, xi_o.at[pl.ds(base + r * half, half)])

    return gather_k


def _unpack_sel(x_u32, par):
    v_lo = jax.lax.bitcast_convert_type(x_u32 << 16, jnp.float32)
    v_hi = jax.lax.bitcast_convert_type((x_u32 >> 16) << 16, jnp.float32)
    return jnp.where(par, v_hi, v_lo)


def _mlp_body(xu_r, xi_r, paru_r, pari_r, bias_r, Wb_r, bb_r,
              W1a_r, W1b_r, W1c_r, b1_r, W2_r, b2_r, W3_r, b3_r, W4_r, b4_r,
              out_r):
    f32 = jnp.float32
    su = _unpack_sel(xu_r[...], paru_r[...] > 0)
    si = _unpack_sel(xi_r[...], pari_r[...] > 0)
    f1 = su[:, :KF]
    f2 = si[:, KF:2 * KF]
    feat = si[:, 2 * KF:3 * KF]
    h = (jnp.dot(f1, W1a_r[...], preferred_element_type=f32)
         + jnp.dot(f2, W1b_r[...], preferred_element_type=f32)
         + jnp.dot(feat, W1c_r[...], preferred_element_type=f32)
         + b1_r[...])
    h = jnp.maximum(h, 0.0)
    h = jnp.maximum(jnp.dot(h, W2_r[...], preferred_element_type=f32) + b2_r[...], 0.0)
    h = jnp.maximum(jnp.dot(h, W3_r[...], preferred_element_type=f32) + b3_r[...], 0.0)
    h4 = jnp.dot(h, W4_r[...], preferred_element_type=f32) + b4_r[...]
    out_r[...] = h4 + bias_r[...] * Wb_r[0, 0] + bb_r[...]


def _mlp_call(xu, xi, paru, pari, bias_feat, Wb, bb,
              W1a, W1b, W1c, b1, W2, b2, W3, b3, W4, b4, *, bm=2048):
    grid = (B // bm,)
    D = 4 * KF

    def row_spec(d):
        return pl.BlockSpec((bm, d), lambda i: (i, 0))

    def full_spec(a):
        return pl.BlockSpec(a.shape, lambda i: (0,) * a.ndim)

    return pl.pallas_call(
        _mlp_body,
        grid=grid,
        in_specs=[
            row_spec(D), row_spec(D), row_spec(1), row_spec(1), row_spec(1),
            full_spec(Wb), full_spec(bb),
            full_spec(W1a), full_spec(W1b), full_spec(W1c), full_spec(b1),
            full_spec(W2), full_spec(b2), full_spec(W3), full_spec(b3),
            full_spec(W4), full_spec(b4),
        ],
        out_specs=row_spec(1),
        out_shape=jax.ShapeDtypeStruct((B, 1), jnp.float32),
    )(xu, xi, paru, pari, bias_feat, Wb, bb,
      W1a, W1b, W1c, b1, W2, b2, W3, b3, W4, b4)


def kernel(user_ids, item_ids, bias_feat, user_table, item_table, gvec,
           Wf, bf, Wb, bb, W1, b1, W2, b2, W3, b3, W4, b4):
    uids = user_ids.reshape(B).astype(jnp.int32)
    iids = item_ids.reshape(B).astype(jnp.int32)
    D = 4 * KF
    M = jnp.zeros((D, D), jnp.float32)
    eye = jnp.eye(KF, dtype=jnp.float32)
    M = M.at[:KF, :KF].set(eye).at[KF:2 * KF, KF:2 * KF].set(eye)
    M = M.at[2 * KF:, 2 * KF:2 * KF + 30].set(Wf)
    brow = jnp.zeros((1, D), jnp.float32).at[0, 2 * KF:2 * KF + 30].set(bf)
    mrow = jnp.zeros((1, D), jnp.float32).at[0, 2 * KF:2 * KF + 30].set(1.0)
    x = _tx_call(user_table.T, item_table.T, gvec.T, M, brow, mrow)
    bn = 4096
    uoff, ioff = uids % bn, iids % bn
    uidx2 = (uids // bn) * (bn // 2) + uoff % (bn // 2)
    iidx2 = (iids // bn) * (bn // 2) + ioff % (bn // 2)
    xu, xi = _sc_gather()(uidx2, iidx2, x)
    paru = (uoff >= bn // 2).astype(jnp.int32).reshape(B, 1)
    pari = (ioff >= bn // 2).astype(jnp.int32).reshape(B, 1)
    W1a, W1b = W1[:KF], W1[KF:2 * KF]
    W1cP = jnp.pad(W1[2 * KF:], ((0, KF - 30), (0, 0)))
    return _mlp_call(
        xu, xi, paru, pari, bias_feat, Wb, bb.reshape(1, 1),
        W1a, W1b, W1cP, b1.reshape(1, -1), W2, b2.reshape(1, -1),
        W3, b3.reshape(1, -1), W4, b4.reshape(1, 1))


# speedup vs baseline: 1.2213x; 1.2213x over previous
import functools

import jax
import jax.numpy as jnp
from jax import lax
from jax.experimental import pallas as pl
from jax.experimental.pallas import tpu as pltpu
from jax.experimental.pallas import tpu_sc as plsc

B = 16384
KF = 32
FL = 64
N = 1000000


def _dot0(a, b):
    return jax.lax.dot_general(a, b, (((0,), (0,)), ((), ())),
                               preferred_element_type=jnp.float32)


def _tx_body(u_r, i_r, g_r, M_r, brow_r, mrow_r, x_r):
    cat = jnp.concatenate([u_r[...], i_r[...], g_r[...]], axis=0)
    h = _dot0(cat, M_r[...]) + brow_r[...]
    x_r[...] = jnp.where(mrow_r[...] > 0, jnp.maximum(h, 0.0), h)


def _tx_call(utT, itT, gT, M, brow, mrow, *, bn=8192):
    grid = (pl.cdiv(N, bn),)
    return pl.pallas_call(
        _tx_body,
        grid=grid,
        in_specs=[
            pl.BlockSpec((KF, bn), lambda i: (0, i)),
            pl.BlockSpec((KF, bn), lambda i: (0, i)),
            pl.BlockSpec((FL, bn), lambda i: (0, i)),
            pl.BlockSpec((4 * KF, 4 * KF), lambda i: (0, 0)),
            pl.BlockSpec((1, 4 * KF), lambda i: (0, 0)),
            pl.BlockSpec((1, 4 * KF), lambda i: (0, 0)),
        ],
        out_specs=pl.BlockSpec((bn, 4 * KF), lambda i: (i, 0)),
        out_shape=jax.ShapeDtypeStruct((N, 4 * KF), jnp.float32),
    )(utT, itT, gT, M, brow, mrow)


@functools.cache
def _sc_gather():
    info = plsc.get_sparse_core_info()
    nw = info.num_cores * info.num_subcores
    bpw = B // nw
    half = bpw // 2
    D = 2 * KF + FL
    mesh = plsc.VectorSubcoreMesh(core_axis_name="c", subcore_axis_name="s")

    @functools.partial(
        pl.kernel,
        mesh=mesh,
        out_type=[
            jax.ShapeDtypeStruct((B, D), jnp.float32),
            jax.ShapeDtypeStruct((B, D), jnp.float32),
        ],
        scratch_types=[
            pltpu.VMEM((bpw,), jnp.int32),
            pltpu.VMEM((bpw,), jnp.int32),
            pltpu.VMEM((half, D), jnp.float32),
            pltpu.VMEM((half, D), jnp.float32),
            pltpu.SemaphoreType.DMA,
            pltpu.SemaphoreType.DMA,
        ],
    )
    def gather_k(uids, iids, x, xu_o, xi_o,
                 uidx, iidx, buf_a, buf_b, sem_a, sem_b):
        wid = lax.axis_index("s") * info.num_cores + lax.axis_index("c")
        base = wid * bpw
        pltpu.sync_copy(uids.at[pl.ds(base, bpw)], uidx)
        pltpu.sync_copy(iids.at[pl.ds(base, bpw)], iidx)

        @pl.loop(0, 2)
        def _(r):
            ca = pltpu.async_copy(
                x.at[uidx.at[pl.ds(r * half, half)]], buf_a, sem_a)
            cb = pltpu.async_copy(
                x.at[iidx.at[pl.ds(r * half, half)]], buf_b, sem_b)
            ca.wait()
            cb.wait()
            pltpu.sync_copy(buf_a, xu_o.at[pl.ds(base + r * half, half)])
            pltpu.sync_copy(buf_b, xi_o.at[pl.ds(base + r * half, half)])

    return gather_k


def _mlp_body(xu_r, xi_r, bias_r, Wb_r, bb_r,
              W1a_r, W1b_r, W1c_r, b1_r, W2_r, b2_r, W3_r, b3_r, W4_r, b4_r,
              out_r):
    f32 = jnp.float32
    f1 = xu_r[:, :KF]
    f2 = xi_r[:, KF:2 * KF]
    feat = xi_r[:, 2 * KF:3 * KF]
    h = (jnp.dot(f1, W1a_r[...], preferred_element_type=f32)
         + jnp.dot(f2, W1b_r[...], preferred_element_type=f32)
         + jnp.dot(feat, W1c_r[...], preferred_element_type=f32)
         + b1_r[...])
    h = jnp.maximum(h, 0.0)
    h = jnp.maximum(jnp.dot(h, W2_r[...], preferred_element_type=f32) + b2_r[...], 0.0)
    h = jnp.maximum(jnp.dot(h, W3_r[...], preferred_element_type=f32) + b3_r[...], 0.0)
    h4 = jnp.dot(h, W4_r[...], preferred_element_type=f32) + b4_r[...]
    out_r[...] = h4 + bias_r[...] * Wb_r[0, 0] + bb_r[...]


def _mlp_call(xu, xi, bias_feat, Wb, bb,
              W1a, W1b, W1c, b1, W2, b2, W3, b3, W4, b4, *, bm=2048):
    grid = (B // bm,)
    D = 4 * KF

    def row_spec(d):
        return pl.BlockSpec((bm, d), lambda i: (i, 0))

    def full_spec(a):
        return pl.BlockSpec(a.shape, lambda i: (0,) * a.ndim)

    return pl.pallas_call(
        _mlp_body,
        grid=grid,
        in_specs=[
            row_spec(D), row_spec(D), row_spec(1),
            full_spec(Wb), full_spec(bb),
            full_spec(W1a), full_spec(W1b), full_spec(W1c), full_spec(b1),
            full_spec(W2), full_spec(b2), full_spec(W3), full_spec(b3),
            full_spec(W4), full_spec(b4),
        ],
        out_specs=row_spec(1),
        out_shape=jax.ShapeDtypeStruct((B, 1), jnp.float32),
    )(xu, xi, bias_feat, Wb, bb,
      W1a, W1b, W1c, b1, W2, b2, W3, b3, W4, b4)


def kernel(user_ids, item_ids, bias_feat, user_table, item_table, gvec,
           Wf, bf, Wb, bb, W1, b1, W2, b2, W3, b3, W4, b4):
    uids = user_ids.reshape(B).astype(jnp.int32)
    iids = item_ids.reshape(B).astype(jnp.int32)
    D = 4 * KF
    M = jnp.zeros((D, D), jnp.float32)
    eye = jnp.eye(KF, dtype=jnp.float32)
    M = M.at[:KF, :KF].set(eye).at[KF:2 * KF, KF:2 * KF].set(eye)
    M = M.at[2 * KF:, 2 * KF:2 * KF + 30].set(Wf)
    brow = jnp.zeros((1, D), jnp.float32).at[0, 2 * KF:2 * KF + 30].set(bf)
    mrow = jnp.zeros((1, D), jnp.float32).at[0, 2 * KF:2 * KF + 30].set(1.0)
    x = _tx_call(user_table.T, item_table.T, gvec.T, M, brow, mrow)
    xu, xi = _sc_gather()(uids, iids, x)
    W1a, W1b = W1[:KF], W1[KF:2 * KF]
    W1cP = jnp.pad(W1[2 * KF:], ((0, KF - 30), (0, 0)))
    return _mlp_call(
        xu, xi, bias_feat, Wb, bb.reshape(1, 1),
        W1a, W1b, W1cP, b1.reshape(1, -1), W2, b2.reshape(1, -1),
        W3, b3.reshape(1, -1), W4, b4.reshape(1, 1))


# speedup vs baseline: 1.2407x; 1.0159x over previous
import functools

import jax
import jax.numpy as jnp
from jax import lax
from jax.experimental import pallas as pl
from jax.experimental.pallas import tpu as pltpu
from jax.experimental.pallas import tpu_sc as plsc

B = 16384
KF = 32
FL = 64
N = 1000000


def _dot0(a, b):
    return jax.lax.dot_general(a, b, (((0,), (0,)), ((), ())),
                               preferred_element_type=jnp.float32)


def _tx_body(u_r, i_r, g_r, M_r, brow_r, mrow_r, x_r):
    cat = jnp.concatenate([u_r[...], i_r[...], g_r[...]], axis=0)
    h = _dot0(cat, M_r[...]) + brow_r[...]
    x_r[...] = jnp.where(mrow_r[...] > 0, jnp.maximum(h, 0.0), h)


def _tx_call(utT, itT, gT, M, brow, mrow, *, bn=16384):
    grid = (pl.cdiv(N, bn),)
    return pl.pallas_call(
        _tx_body,
        grid=grid,
        in_specs=[
            pl.BlockSpec((KF, bn), lambda i: (0, i)),
            pl.BlockSpec((KF, bn), lambda i: (0, i)),
            pl.BlockSpec((FL, bn), lambda i: (0, i)),
            pl.BlockSpec((4 * KF, 4 * KF), lambda i: (0, 0)),
            pl.BlockSpec((1, 4 * KF), lambda i: (0, 0)),
            pl.BlockSpec((1, 4 * KF), lambda i: (0, 0)),
        ],
        out_specs=pl.BlockSpec((bn, 4 * KF), lambda i: (i, 0)),
        out_shape=jax.ShapeDtypeStruct((N, 4 * KF), jnp.float32),
    )(utT, itT, gT, M, brow, mrow)


@functools.cache
def _sc_gather():
    info = plsc.get_sparse_core_info()
    nw = info.num_cores * info.num_subcores
    bpw = B // nw
    half = bpw // 2
    D = 2 * KF + FL
    mesh = plsc.VectorSubcoreMesh(core_axis_name="c", subcore_axis_name="s")

    @functools.partial(
        pl.kernel,
        mesh=mesh,
        out_type=[
            jax.ShapeDtypeStruct((B, D), jnp.float32),
            jax.ShapeDtypeStruct((B, D), jnp.float32),
        ],
        scratch_types=[
            pltpu.VMEM((bpw,), jnp.int32),
            pltpu.VMEM((bpw,), jnp.int32),
            pltpu.VMEM((half, D), jnp.float32),
            pltpu.VMEM((half, D), jnp.float32),
            pltpu.SemaphoreType.DMA,
            pltpu.SemaphoreType.DMA,
        ],
    )
    def gather_k(uids, iids, x, xu_o, xi_o,
                 uidx, iidx, buf_a, buf_b, sem_a, sem_b):
        wid = lax.axis_index("s") * info.num_cores + lax.axis_index("c")
        base = wid * bpw
        pltpu.sync_copy(uids.at[pl.ds(base, bpw)], uidx)
        pltpu.sync_copy(iids.at[pl.ds(base, bpw)], iidx)

        @pl.loop(0, 2)
        def _(r):
            ca = pltpu.async_copy(
                x.at[uidx.at[pl.ds(r * half, half)]], buf_a, sem_a)
            cb = pltpu.async_copy(
                x.at[iidx.at[pl.ds(r * half, half)]], buf_b, sem_b)
            ca.wait()
            cb.wait()
            pltpu.sync_copy(buf_a, xu_o.at[pl.ds(base + r * half, half)])
            pltpu.sync_copy(buf_b, xi_o.at[pl.ds(base + r * half, half)])

    return gather_k


def _mlp_body(xu_r, xi_r, bias_r, Wb_r, bb_r,
              W1a_r, W1b_r, W1c_r, b1_r, W2_r, b2_r, W3_r, b3_r, W4_r, b4_r,
              out_r):
    f32 = jnp.float32
    f1 = xu_r[:, :KF]
    f2 = xi_r[:, KF:2 * KF]
    feat = xi_r[:, 2 * KF:3 * KF]
    h = (jnp.dot(f1, W1a_r[...], preferred_element_type=f32)
         + jnp.dot(f2, W1b_r[...], preferred_element_type=f32)
         + jnp.dot(feat, W1c_r[...], preferred_element_type=f32)
         + b1_r[...])
    h = jnp.maximum(h, 0.0)
    h = jnp.maximum(jnp.dot(h, W2_r[...], preferred_element_type=f32) + b2_r[...], 0.0)
    h = jnp.maximum(jnp.dot(h, W3_r[...], preferred_element_type=f32) + b3_r[...], 0.0)
    h4 = jnp.dot(h, W4_r[...], preferred_element_type=f32) + b4_r[...]
    out_r[...] = h4 + bias_r[...] * Wb_r[0, 0] + bb_r[...]


def _mlp_call(xu, xi, bias_feat, Wb, bb,
              W1a, W1b, W1c, b1, W2, b2, W3, b3, W4, b4, *, bm=2048):
    grid = (B // bm,)
    D = 4 * KF

    def row_spec(d):
        return pl.BlockSpec((bm, d), lambda i: (i, 0))

    def full_spec(a):
        return pl.BlockSpec(a.shape, lambda i: (0,) * a.ndim)

    return pl.pallas_call(
        _mlp_body,
        grid=grid,
        in_specs=[
            row_spec(D), row_spec(D), row_spec(1),
            full_spec(Wb), full_spec(bb),
            full_spec(W1a), full_spec(W1b), full_spec(W1c), full_spec(b1),
            full_spec(W2), full_spec(b2), full_spec(W3), full_spec(b3),
            full_spec(W4), full_spec(b4),
        ],
        out_specs=row_spec(1),
        out_shape=jax.ShapeDtypeStruct((B, 1), jnp.float32),
    )(xu, xi, bias_feat, Wb, bb,
      W1a, W1b, W1c, b1, W2, b2, W3, b3, W4, b4)


def kernel(user_ids, item_ids, bias_feat, user_table, item_table, gvec,
           Wf, bf, Wb, bb, W1, b1, W2, b2, W3, b3, W4, b4):
    uids = user_ids.reshape(B).astype(jnp.int32)
    iids = item_ids.reshape(B).astype(jnp.int32)
    D = 4 * KF
    M = jnp.zeros((D, D), jnp.float32)
    eye = jnp.eye(KF, dtype=jnp.float32)
    M = M.at[:KF, :KF].set(eye).at[KF:2 * KF, KF:2 * KF].set(eye)
    M = M.at[2 * KF:, 2 * KF:2 * KF + 30].set(Wf)
    brow = jnp.zeros((1, D), jnp.float32).at[0, 2 * KF:2 * KF + 30].set(bf)
    mrow = jnp.zeros((1, D), jnp.float32).at[0, 2 * KF:2 * KF + 30].set(1.0)
    x = _tx_call(user_table.T, item_table.T, gvec.T, M, brow, mrow)
    xu, xi = _sc_gather()(uids, iids, x)
    W1a, W1b = W1[:KF], W1[KF:2 * KF]
    W1cP = jnp.pad(W1[2 * KF:], ((0, KF - 30), (0, 0)))
    return _mlp_call(
        xu, xi, bias_feat, Wb, bb.reshape(1, 1),
        W1a, W1b, W1cP, b1.reshape(1, -1), W2, b2.reshape(1, -1),
        W3, b3.reshape(1, -1), W4, b4.reshape(1, 1))


# speedup vs baseline: 1.3068x; 1.0533x over previous
import functools

import jax
import jax.numpy as jnp
from jax import lax
from jax.experimental import pallas as pl
from jax.experimental.pallas import tpu as pltpu
from jax.experimental.pallas import tpu_sc as plsc

B = 16384
KF = 32
FL = 64
N = 1000000


def _dot0(a, b):
    return jax.lax.dot_general(a, b, (((0,), (0,)), ((), ())),
                               preferred_element_type=jnp.float32)


def _round_bf16_bits(v):
    b = jax.lax.bitcast_convert_type(v, jnp.uint32)
    return (b + 0x7FFF + ((b >> 16) & 1)) >> 16


def _tx_body(u_r, i_r, g_r, M_r, brow_r, mrow_r, x_r):
    bn = u_r.shape[1]
    cat = jnp.concatenate([u_r[...], i_r[...], g_r[...]], axis=0)
    h = _dot0(cat, M_r[...]) + brow_r[...]
    x = jnp.where(mrow_r[...] > 0, jnp.maximum(h, 0.0), h)
    be = _round_bf16_bits(x[:bn // 2])
    bo = _round_bf16_bits(x[bn // 2:])
    x_r[...] = be | (bo << 16)


def _tx_call(utT, itT, gT, M, brow, mrow, *, bn=16384):
    grid = (pl.cdiv(N, bn),)
    return pl.pallas_call(
        _tx_body,
        grid=grid,
        in_specs=[
            pl.BlockSpec((KF, bn), lambda i: (0, i)),
            pl.BlockSpec((KF, bn), lambda i: (0, i)),
            pl.BlockSpec((FL, bn), lambda i: (0, i)),
            pl.BlockSpec((4 * KF, 4 * KF), lambda i: (0, 0)),
            pl.BlockSpec((1, 4 * KF), lambda i: (0, 0)),
            pl.BlockSpec((1, 4 * KF), lambda i: (0, 0)),
        ],
        out_specs=pl.BlockSpec((bn // 2, 4 * KF), lambda i: (i, 0)),
        out_shape=jax.ShapeDtypeStruct(
            (pl.cdiv(N, bn) * (bn // 2), 4 * KF), jnp.uint32),
    )(utT, itT, gT, M, brow, mrow)


@functools.cache
def _sc_gather():
    info = plsc.get_sparse_core_info()
    nw = info.num_cores * info.num_subcores
    bpw = B // nw
    half = bpw // 2
    D = 2 * KF + FL
    mesh = plsc.VectorSubcoreMesh(core_axis_name="c", subcore_axis_name="s")

    @functools.partial(
        pl.kernel,
        mesh=mesh,
        out_type=[
            jax.ShapeDtypeStruct((B, D), jnp.uint32),
            jax.ShapeDtypeStruct((B, D), jnp.uint32),
        ],
        scratch_types=[
            pltpu.VMEM((bpw,), jnp.int32),
            pltpu.VMEM((bpw,), jnp.int32),
            pltpu.VMEM((half, D), jnp.uint32),
            pltpu.VMEM((half, D), jnp.uint32),
            pltpu.SemaphoreType.DMA,
            pltpu.SemaphoreType.DMA,
        ],
    )
    def gather_k(uids, iids, x, xu_o, xi_o,
                 uidx, iidx, buf_a, buf_b, sem_a, sem_b):
        wid = lax.axis_index("s") * info.num_cores + lax.axis_index("c")
        base = wid * bpw
        pltpu.sync_copy(uids.at[pl.ds(base, bpw)], uidx)
        pltpu.sync_copy(iids.at[pl.ds(base, bpw)], iidx)

        @pl.loop(0, 2)
        def _(r):
            ca = pltpu.async_copy(
                x.at[uidx.at[pl.ds(r * half, half)]], buf_a, sem_a)
            cb = pltpu.async_copy(
                x.at[iidx.at[pl.ds(r * half, half)]], buf_b, sem_b)
            ca.wait()
            cb.wait()
            pltpu.sync_copy(buf_a, xu_o.at[pl.ds(base + r * half, half)])
            pltpu.sync_copy(buf_b, xi_o.at[pl.ds(base + r * half, half)])

    return gather_k


def _unpack_sel(x_u32, par):
    v_lo = jax.lax.bitcast_convert_type(x_u32 << 16, jnp.float32)
    v_hi = jax.lax.bitcast_convert_type((x_u32 >> 16) << 16, jnp.float32)
    return jnp.where(par, v_hi, v_lo)


def _mlp_body(xu_r, xi_r, paru_r, pari_r, bias_r, Wb_r, bb_r,
              W1a_r, W1b_r, W1c_r, b1_r, W2_r, b2_r, W3_r, b3_r, W4_r, b4_r,
              out_r):
    f32 = jnp.float32
    su = _unpack_sel(xu_r[...], paru_r[...] > 0)
    si = _unpack_sel(xi_r[...], pari_r[...] > 0)
    f1 = su[:, :KF]
    f2 = si[:, KF:2 * KF]
    feat = si[:, 2 * KF:3 * KF]
    h = (jnp.dot(f1, W1a_r[...], preferred_element_type=f32)
         + jnp.dot(f2, W1b_r[...], preferred_element_type=f32)
         + jnp.dot(feat, W1c_r[...], preferred_element_type=f32)
         + b1_r[...])
    h = jnp.maximum(h, 0.0)
    h = jnp.maximum(jnp.dot(h, W2_r[...], preferred_element_type=f32) + b2_r[...], 0.0)
    h = jnp.maximum(jnp.dot(h, W3_r[...], preferred_element_type=f32) + b3_r[...], 0.0)
    h4 = jnp.dot(h, W4_r[...], preferred_element_type=f32) + b4_r[...]
    out_r[...] = h4 + bias_r[...] * Wb_r[0, 0] + bb_r[...]


def _mlp_call(xu, xi, paru, pari, bias_feat, Wb, bb,
              W1a, W1b, W1c, b1, W2, b2, W3, b3, W4, b4, *, bm=2048):
    grid = (B // bm,)
    D = 4 * KF

    def row_spec(d):
        return pl.BlockSpec((bm, d), lambda i: (i, 0))

    def full_spec(a):
        return pl.BlockSpec(a.shape, lambda i: (0,) * a.ndim)

    return pl.pallas_call(
        _mlp_body,
        grid=grid,
        in_specs=[
            row_spec(D), row_spec(D), row_spec(1), row_spec(1), row_spec(1),
            full_spec(Wb), full_spec(bb),
            full_spec(W1a), full_spec(W1b), full_spec(W1c), full_spec(b1),
            full_spec(W2), full_spec(b2), full_spec(W3), full_spec(b3),
            full_spec(W4), full_spec(b4),
        ],
        out_specs=row_spec(1),
        out_shape=jax.ShapeDtypeStruct((B, 1), jnp.float32),
    )(xu, xi, paru, pari, bias_feat, Wb, bb,
      W1a, W1b, W1c, b1, W2, b2, W3, b3, W4, b4)


def kernel(user_ids, item_ids, bias_feat, user_table, item_table, gvec,
           Wf, bf, Wb, bb, W1, b1, W2, b2, W3, b3, W4, b4):
    uids = user_ids.reshape(B).astype(jnp.int32)
    iids = item_ids.reshape(B).astype(jnp.int32)
    D = 4 * KF
    M = jnp.zeros((D, D), jnp.float32)
    eye = jnp.eye(KF, dtype=jnp.float32)
    M = M.at[:KF, :KF].set(eye).at[KF:2 * KF, KF:2 * KF].set(eye)
    M = M.at[2 * KF:, 2 * KF:2 * KF + 30].set(Wf)
    brow = jnp.zeros((1, D), jnp.float32).at[0, 2 * KF:2 * KF + 30].set(bf)
    mrow = jnp.zeros((1, D), jnp.float32).at[0, 2 * KF:2 * KF + 30].set(1.0)
    x = _tx_call(user_table.T, item_table.T, gvec.T, M, brow, mrow)
    bn = 16384
    uoff, ioff = uids % bn, iids % bn
    uidx2 = (uids // bn) * (bn // 2) + uoff % (bn // 2)
    iidx2 = (iids // bn) * (bn // 2) + ioff % (bn // 2)
    xu, xi = _sc_gather()(uidx2, iidx2, x)
    paru = (uoff >= bn // 2).astype(jnp.int32).reshape(B, 1)
    pari = (ioff >= bn // 2).astype(jnp.int32).reshape(B, 1)
    W1a, W1b = W1[:KF], W1[KF:2 * KF]
    W1cP = jnp.pad(W1[2 * KF:], ((0, KF - 30), (0, 0)))
    return _mlp_call(
        xu, xi, paru, pari, bias_feat, Wb, bb.reshape(1, 1),
        W1a, W1b, W1cP, b1.reshape(1, -1), W2, b2.reshape(1, -1),
        W3, b3.reshape(1, -1), W4, b4.reshape(1, 1))


# speedup vs baseline: 1.3400x; 1.0254x over previous
import functools

import jax
import jax.numpy as jnp
from jax import lax
from jax.experimental import pallas as pl
from jax.experimental.pallas import tpu as pltpu
from jax.experimental.pallas import tpu_sc as plsc

B = 16384
KF = 32
FL = 64
N = 1000000


def _dot0(a, b):
    return jax.lax.dot_general(a, b, (((0,), (0,)), ((), ())),
                               preferred_element_type=jnp.float32)


def _round_bf16_bits(v):
    b = jax.lax.bitcast_convert_type(v, jnp.uint32)
    return (b + 0x7FFF + ((b >> 16) & 1)) >> 16


def _tx_body(u_r, i_r, g_r, M_r, brow_r, mrow_r, x_r):
    bn = u_r.shape[1]
    cat = jnp.concatenate([u_r[...], i_r[...], g_r[...]], axis=0)
    h = _dot0(cat, M_r[...]) + brow_r[...]
    x = jnp.where(mrow_r[...] > 0, jnp.maximum(h, 0.0), h)
    be = _round_bf16_bits(x[:bn // 2])
    bo = _round_bf16_bits(x[bn // 2:])
    x_r[...] = be | (bo << 16)


def _tx_call(utT, itT, gT, M, brow, mrow, *, bn=32768):
    grid = (pl.cdiv(N, bn),)
    return pl.pallas_call(
        _tx_body,
        grid=grid,
        in_specs=[
            pl.BlockSpec((KF, bn), lambda i: (0, i)),
            pl.BlockSpec((KF, bn), lambda i: (0, i)),
            pl.BlockSpec((FL, bn), lambda i: (0, i)),
            pl.BlockSpec((4 * KF, 4 * KF), lambda i: (0, 0)),
            pl.BlockSpec((1, 4 * KF), lambda i: (0, 0)),
            pl.BlockSpec((1, 4 * KF), lambda i: (0, 0)),
        ],
        out_specs=pl.BlockSpec((bn // 2, 4 * KF), lambda i: (i, 0)),
        out_shape=jax.ShapeDtypeStruct(
            (pl.cdiv(N, bn) * (bn // 2), 4 * KF), jnp.uint32),
    )(utT, itT, gT, M, brow, mrow)


@functools.cache
def _sc_gather():
    info = plsc.get_sparse_core_info()
    nw = info.num_cores * info.num_subcores
    bpw = B // nw
    half = bpw // 2
    D = 2 * KF + FL
    mesh = plsc.VectorSubcoreMesh(core_axis_name="c", subcore_axis_name="s")

    @functools.partial(
        pl.kernel,
        mesh=mesh,
        out_type=[
            jax.ShapeDtypeStruct((B, D), jnp.uint32),
            jax.ShapeDtypeStruct((B, D), jnp.uint32),
        ],
        scratch_types=[
            pltpu.VMEM((bpw,), jnp.int32),
            pltpu.VMEM((bpw,), jnp.int32),
            pltpu.VMEM((half, D), jnp.uint32),
            pltpu.VMEM((half, D), jnp.uint32),
            pltpu.SemaphoreType.DMA,
            pltpu.SemaphoreType.DMA,
        ],
    )
    def gather_k(uids, iids, x, xu_o, xi_o,
                 uidx, iidx, buf_a, buf_b, sem_a, sem_b):
        wid = lax.axis_index("s") * info.num_cores + lax.axis_index("c")
        base = wid * bpw
        pltpu.sync_copy(uids.at[pl.ds(base, bpw)], uidx)
        pltpu.sync_copy(iids.at[pl.ds(base, bpw)], iidx)

        @pl.loop(0, 2)
        def _(r):
            ca = pltpu.async_copy(
                x.at[uidx.at[pl.ds(r * half, half)]], buf_a, sem_a)
            cb = pltpu.async_copy(
                x.at[iidx.at[pl.ds(r * half, half)]], buf_b, sem_b)
            ca.wait()
            cb.wait()
            pltpu.sync_copy(buf_a, xu_o.at[pl.ds(base + r * half, half)])
            pltpu.sync_copy(buf_b, xi_o.at[pl.ds(base + r * half, half)])

    return gather_k


def _unpack_sel(x_u32, par):
    v_lo = jax.lax.bitcast_convert_type(x_u32 << 16, jnp.float32)
    v_hi = jax.lax.bitcast_convert_type((x_u32 >> 16) << 16, jnp.float32)
    return jnp.where(par, v_hi, v_lo)


def _mlp_body(xu_r, xi_r, paru_r, pari_r, bias_r, Wb_r, bb_r,
              W1a_r, W1b_r, W1c_r, b1_r, W2_r, b2_r, W3_r, b3_r, W4_r, b4_r,
              out_r):
    f32 = jnp.float32
    su = _unpack_sel(xu_r[...], paru_r[...] > 0)
    si = _unpack_sel(xi_r[...], pari_r[...] > 0)
    f1 = su[:, :KF]
    f2 = si[:, KF:2 * KF]
    feat = si[:, 2 * KF:3 * KF]
    h = (jnp.dot(f1, W1a_r[...], preferred_element_type=f32)
         + jnp.dot(f2, W1b_r[...], preferred_element_type=f32)
         + jnp.dot(feat, W1c_r[...], preferred_element_type=f32)
         + b1_r[...])
    h = jnp.maximum(h, 0.0)
    h = jnp.maximum(jnp.dot(h, W2_r[...], preferred_element_type=f32) + b2_r[...], 0.0)
    h = jnp.maximum(jnp.dot(h, W3_r[...], preferred_element_type=f32) + b3_r[...], 0.0)
    h4 = jnp.dot(h, W4_r[...], preferred_element_type=f32) + b4_r[...]
    out_r[...] = h4 + bias_r[...] * Wb_r[0, 0] + bb_r[...]


def _mlp_call(xu, xi, paru, pari, bias_feat, Wb, bb,
              W1a, W1b, W1c, b1, W2, b2, W3, b3, W4, b4, *, bm=2048):
    grid = (B // bm,)
    D = 4 * KF

    def row_spec(d):
        return pl.BlockSpec((bm, d), lambda i: (i, 0))

    def full_spec(a):
        return pl.BlockSpec(a.shape, lambda i: (0,) * a.ndim)

    return pl.pallas_call(
        _mlp_body,
        grid=grid,
        in_specs=[
            row_spec(D), row_spec(D), row_spec(1), row_spec(1), row_spec(1),
            full_spec(Wb), full_spec(bb),
            full_spec(W1a), full_spec(W1b), full_spec(W1c), full_spec(b1),
            full_spec(W2), full_spec(b2), full_spec(W3), full_spec(b3),
            full_spec(W4), full_spec(b4),
        ],
        out_specs=row_spec(1),
        out_shape=jax.ShapeDtypeStruct((B, 1), jnp.float32),
    )(xu, xi, paru, pari, bias_feat, Wb, bb,
      W1a, W1b, W1c, b1, W2, b2, W3, b3, W4, b4)


def kernel(user_ids, item_ids, bias_feat, user_table, item_table, gvec,
           Wf, bf, Wb, bb, W1, b1, W2, b2, W3, b3, W4, b4):
    uids = user_ids.reshape(B).astype(jnp.int32)
    iids = item_ids.reshape(B).astype(jnp.int32)
    D = 4 * KF
    M = jnp.zeros((D, D), jnp.float32)
    eye = jnp.eye(KF, dtype=jnp.float32)
    M = M.at[:KF, :KF].set(eye).at[KF:2 * KF, KF:2 * KF].set(eye)
    M = M.at[2 * KF:, 2 * KF:2 * KF + 30].set(Wf)
    brow = jnp.zeros((1, D), jnp.float32).at[0, 2 * KF:2 * KF + 30].set(bf)
    mrow = jnp.zeros((1, D), jnp.float32).at[0, 2 * KF:2 * KF + 30].set(1.0)
    x = _tx_call(user_table.T, item_table.T, gvec.T, M, brow, mrow)
    bn = 32768
    uoff, ioff = uids % bn, iids % bn
    uidx2 = (uids // bn) * (bn // 2) + uoff % (bn // 2)
    iidx2 = (iids // bn) * (bn // 2) + ioff % (bn // 2)
    xu, xi = _sc_gather()(uidx2, iidx2, x)
    paru = (uoff >= bn // 2).astype(jnp.int32).reshape(B, 1)
    pari = (ioff >= bn // 2).astype(jnp.int32).reshape(B, 1)
    W1a, W1b = W1[:KF], W1[KF:2 * KF]
    W1cP = jnp.pad(W1[2 * KF:], ((0, KF - 30), (0, 0)))
    return _mlp_call(
        xu, xi, paru, pari, bias_feat, Wb, bb.reshape(1, 1),
        W1a, W1b, W1cP, b1.reshape(1, -1), W2, b2.reshape(1, -1),
        W3, b3.reshape(1, -1), W4, b4.reshape(1, 1))
